# baseline (speedup 1.0000x reference)
import jax
import jax.numpy as jnp
from jax import lax
from jax.experimental import pallas as pl
from jax.experimental.pallas import tpu as pltpu
from jax.experimental.pallas import tpu_sc as plsc

N = 10000
E = 320000
D = 128
EF = 16
H = 4

NC = 2
NS = 16
NW = NC * NS
ECH = E // NS
CHUNK = E // NW
B3 = 1000
B4 = 40
ACC_R = 10240
ROWS_PT = ACC_R // NS

f32 = jnp.float32
i32 = jnp.int32

_sc_mesh = dict(
    mesh=plsc.VectorSubcoreMesh(
        core_axis_name="c", subcore_axis_name="s", num_cores=NC,
        num_subcores=NS),
    compiler_params=pltpu.CompilerParams(needs_layout_passes=False),
)


def _node_dense_body(x_ref, wd01_ref, wd23_ref, ws01_ref, ws23_ref,
                     ww0_ref, ww1_ref, bw0_ref, bw1_ref,
                     xw_ref, ad_ref, as_ref):
    xb = x_ref[...]
    xw_ref[0] = jnp.dot(xb, ww0_ref[...], preferred_element_type=f32) + bw0_ref[...]
    xw_ref[1] = jnp.dot(xb, ww1_ref[...], preferred_element_type=f32) + bw1_ref[...]
    ad_ref[0] = jnp.dot(xb, wd01_ref[...], preferred_element_type=f32)
    ad_ref[1] = jnp.dot(xb, wd23_ref[...], preferred_element_type=f32)
    as_ref[0] = jnp.dot(xb, ws01_ref[...], preferred_element_type=f32)
    as_ref[1] = jnp.dot(xb, ws23_ref[...], preferred_element_type=f32)


_node_dense = pl.pallas_call(
    _node_dense_body,
    grid=(10,),
    in_specs=[
        pl.BlockSpec((1000, D), lambda i: (i, 0)),
        pl.BlockSpec((D, 2), lambda i: (0, 0)),
        pl.BlockSpec((D, 2), lambda i: (0, 0)),
        pl.BlockSpec((D, 2), lambda i: (0, 0)),
        pl.BlockSpec((D, 2), lambda i: (0, 0)),
        pl.BlockSpec((D, H * D // 2), lambda i: (0, 0)),
        pl.BlockSpec((D, H * D // 2), lambda i: (0, 0)),
        pl.BlockSpec((1, H * D // 2), lambda i: (0, 0)),
        pl.BlockSpec((1, H * D // 2), lambda i: (0, 0)),
    ],
    out_specs=[
        pl.BlockSpec((NC, 1000, H * D // 2), lambda i: (0, i, 0)),
        pl.BlockSpec((NC, 1000, 2), lambda i: (0, i, 0)),
        pl.BlockSpec((NC, 1000, 2), lambda i: (0, i, 0)),
    ],
    out_shape=[
        jax.ShapeDtypeStruct((NC, N, H * D // 2), f32),
        jax.ShapeDtypeStruct((NC, N, 2), f32),
        jax.ShapeDtypeStruct((NC, N, 2), f32),
    ],
)


def _edge_dense_body(ea_ref, wrel_ref, brel_ref, g_ref, b_ref,
                     w301_ref, w323_ref, batt_ref, ue_ref):
    w = jnp.dot(ea_ref[...], wrel_ref[...], preferred_element_type=f32) + brel_ref[...]
    mu = jnp.mean(w, axis=-1, keepdims=True)
    var = jnp.mean((w - mu) ** 2, axis=-1, keepdims=True)
    w = (w - mu) * lax.rsqrt(var + 1e-5) * g_ref[...] + b_ref[...]
    w = jnp.maximum(w, 0.0)
    ue_ref[0] = (jnp.dot(w, w301_ref[...], preferred_element_type=f32)
                 + batt_ref[:, 0:2])
    ue_ref[1] = (jnp.dot(w, w323_ref[...], preferred_element_type=f32)
                 + batt_ref[:, 2:4])


_edge_dense = pl.pallas_call(
    _edge_dense_body,
    grid=(E // 1000,),
    in_specs=[
        pl.BlockSpec((1000, EF), lambda i: (i, 0)),
        pl.BlockSpec((EF, D), lambda i: (0, 0)),
        pl.BlockSpec((1, D), lambda i: (0, 0)),
        pl.BlockSpec((1, D), lambda i: (0, 0)),
        pl.BlockSpec((1, D), lambda i: (0, 0)),
        pl.BlockSpec((D, 2), lambda i: (0, 0)),
        pl.BlockSpec((D, 2), lambda i: (0, 0)),
        pl.BlockSpec((1, H), lambda i: (0, 0)),
    ],
    out_specs=pl.BlockSpec((NC, 1000, 2), lambda i: (0, i, 0)),
    out_shape=jax.ShapeDtypeStruct((NC, E, 2), f32),
)


def _sc_pass1(src_h, dst_h, ad_h, as_h, ue_h, p_h, spart_h,
              ad_t, as_t, src_v, dst_v, ue_v, p_v, s_loc):
    cid = lax.axis_index("c")
    sid = lax.axis_index("s")
    wid = cid * NS + sid
    base = sid * ECH
    iota = lax.iota(i32, 16)
    lane_h = lax.bitwise_and(iota, 1)
    lane_e = lax.shift_right_logical(iota, 1)

    noff = cid * (N * 2)
    eoff = cid * (E * 2)
    pltpu.sync_copy(ad_h.at[pl.ds(noff, N * 2)], ad_t)
    pltpu.sync_copy(as_h.at[pl.ds(noff, N * 2)], as_t)

    def zbody(i, c):
        s_loc[pl.ds(i * 16, 16)] = jnp.zeros((16,), f32)
        return c
    lax.fori_loop(0, N * 2 // 16, zbody, 0)

    def blk(b, c):
        eb = base + b * B3
        pltpu.sync_copy(src_h.at[pl.ds(eb, B3)], src_v)
        pltpu.sync_copy(dst_h.at[pl.ds(eb, B3)], dst_v)
        pltpu.sync_copy(ue_h.at[pl.ds(eoff + eb * 2, B3 * 2)], ue_v)

        def inner(i, c2):
            eidx = i * 8 + lane_e
            dstv = plsc.load_gather(dst_v, [eidx])
            srcv = plsc.load_gather(src_v, [eidx])
            fd = dstv * 2 + lane_h
            gd = plsc.load_gather(ad_t, [fd])
            gs = plsc.load_gather(as_t, [srcv * 2 + lane_h])
            u = gd + gs + ue_v[pl.ds(i * 16, 16)]
            u = jnp.where(u >= 0, u, u * 0.2)
            p = jnp.exp(u)
            p_v[pl.ds(i * 16, 16)] = p
            plsc.addupdate_scatter(s_loc, [fd], p)
            return c2
        lax.fori_loop(0, B3 * 2 // 16, inner, 0)
        pltpu.sync_copy(p_v, p_h.at[pl.ds(eoff + eb * 2, B3 * 2)])
        return c
    lax.fori_loop(0, ECH // B3, blk, 0)
    pltpu.sync_copy(s_loc, spart_h.at[pl.ds(wid * (N * 2), N * 2)])


_sc1 = pl.kernel(
    _sc_pass1,
    out_type=[
        jax.ShapeDtypeStruct((NC * E * 2,), f32),
        jax.ShapeDtypeStruct((NW * N * 2,), f32),
    ],
    scratch_types=[
        pltpu.VMEM((N * 2,), f32),
        pltpu.VMEM((N * 2,), f32),
        pltpu.VMEM((B3,), i32),
        pltpu.VMEM((B3,), i32),
        pltpu.VMEM((B3 * 2,), f32),
        pltpu.VMEM((B3 * 2,), f32),
        pltpu.VMEM((N * 2,), f32),
    ],
    **_sc_mesh,
)


def _s_reduce_body(sp_ref, rs_ref):
    s0 = jnp.sum(sp_ref[0:NS], axis=0)
    s1 = jnp.sum(sp_ref[NS:NW], axis=0)
    s = jnp.stack([s0, s1], axis=0)
    rs_ref[...] = jnp.where(s > 0, 1.0 / s, 0.0)


_s_reduce = pl.pallas_call(
    _s_reduce_body,
    grid=(1,),
    in_specs=[pl.BlockSpec((NW, N * 2), lambda i: (0, 0))],
    out_specs=pl.BlockSpec((NC, N * 2), lambda i: (0, 0)),
    out_shape=jax.ShapeDtypeStruct((NC, N * 2), f32),
)


def _sc_alpha(dst_h, p_h, rs_h, al_h, rs_t, dst_v, p_v, al_v):
    cid = lax.axis_index("c")
    sid = lax.axis_index("s")
    base = sid * ECH
    iota = lax.iota(i32, 16)
    lane_h = lax.bitwise_and(iota, 1)
    lane_e = lax.shift_right_logical(iota, 1)

    noff = cid * (N * 2)
    eoff = cid * (E * 2)
    pltpu.sync_copy(rs_h.at[pl.ds(noff, N * 2)], rs_t)

    def blk(b, c):
        eb = base + b * B3
        pltpu.sync_copy(dst_h.at[pl.ds(eb, B3)], dst_v)
        pltpu.sync_copy(p_h.at[pl.ds(eoff + eb * 2, B3 * 2)], p_v)

        def inner(i, c2):
            eidx = i * 8 + lane_e
            dstv = plsc.load_gather(dst_v, [eidx])
            rsv = plsc.load_gather(rs_t, [dstv * 2 + lane_h])
            al_v[pl.ds(i * 16, 16)] = p_v[pl.ds(i * 16, 16)] * rsv
            return c2
        lax.fori_loop(0, B3 * 2 // 16, inner, 0)
        pltpu.sync_copy(al_v, al_h.at[pl.ds(eoff + eb * 2, B3 * 2)])
        return c
    lax.fori_loop(0, ECH // B3, blk, 0)


_sc_al = pl.kernel(
    _sc_alpha,
    out_type=jax.ShapeDtypeStruct((NC * E * 2,), f32),
    scratch_types=[
        pltpu.VMEM((N * 2,), f32),
        pltpu.VMEM((B3,), i32),
        pltpu.VMEM((B3 * 2,), f32),
        pltpu.VMEM((B3 * 2,), f32),
    ],
    **_sc_mesh,
)


HD2 = H * D // 2
FH = D // 2
SUP = 1000
NSUP = ECH // SUP
BPS = SUP // B4


NBUF = 5


def _sc_msg(srcx_h, dst_h, al_h, xw_h, acc_h,
            idx_sb, dst_sb, al0_sb,
            xw_bufs, m_bufs, acc_sh, g_sems, s_sem):
    cid = lax.axis_index("c")
    sid = lax.axis_index("s")
    base = sid * ECH

    for i in range(B4):
        for c in range(D // 16):
            m_bufs[0][i, pl.ds(c * 16, 16)] = jnp.zeros((16,), f32)
    row0 = sid * ROWS_PT

    def zcp(j, c):
        pltpu.sync_copy(m_bufs[0], acc_sh.at[pl.ds(row0 + j * B4, B4)])
        return c
    lax.fori_loop(0, ROWS_PT // B4, zcp, 0)
    plsc.subcore_barrier()

    def eloop(blk, xwv, mv):
        def eb(e, c):
            le2 = (blk * B4 + e) * 2
            a0 = plsc.load_gather(al0_sb, [jnp.full((16,), le2, i32)])
            a1 = plsc.load_gather(al0_sb, [jnp.full((16,), le2 + 1, i32)])
            for c4 in range(D // 16):
                v = a0 * xwv[e, pl.ds(c4 * 16, 16)]
                v = v + a1 * xwv[e, pl.ds(D + c4 * 16, 16)]
                mv[e, pl.ds(c4 * 16, 16)] = v
            return c
        lax.fori_loop(0, B4, eb, 0)

    def super_body(s, c):
        off = base + s * SUP
        pltpu.sync_copy(al_h.at[pl.ds(cid * (E * 2) + off * 2, SUP * 2)],
                        al0_sb)

        def blkf(b, c2):
            eb = off + b * B4
            pltpu.sync_copy(srcx_h.at[pl.ds(cid * E + eb, B4)], idx_sb)
            pltpu.sync_copy(dst_h.at[pl.ds(eb, B4)], dst_sb)
            g = pltpu.async_copy(xw_h.at[idx_sb], xw_bufs[0], g_sems[0])
            g.wait()
            eloop(b, xw_bufs[0], m_bufs[0])
            pltpu.sync_copy(m_bufs[0], acc_sh.at[dst_sb], add=True)
            return c2
        lax.fori_loop(0, BPS, blkf, 0)
        return c
    lax.fori_loop(0, NSUP, super_body, 0)
    plsc.subcore_barrier()
    pltpu.sync_copy(acc_sh.at[pl.ds(row0, ROWS_PT)],
                    acc_h.at[cid, pl.ds(row0, ROWS_PT)])


_sc_m = pl.kernel(
    _sc_msg,
    out_type=jax.ShapeDtypeStruct((NC, ACC_R, D), f32),
    scratch_types=[
        pltpu.VMEM((B4,), i32),
        pltpu.VMEM((B4,), i32),
        pltpu.VMEM((SUP * 2,), f32),
        [pltpu.VMEM((B4, HD2), f32)] * 1,
        [pltpu.VMEM((B4, D), f32)] * 1,
        pltpu.VMEM_SHARED((ACC_R, D), f32),
        [pltpu.SemaphoreType.DMA] * 1,
        pltpu.SemaphoreType.DMA,
    ],
    **_sc_mesh,
)


def _final_body(acc_ref, x_ref, rs_ref, pa_ref, out_ref):
    s = (acc_ref[0] + acc_ref[1]) * (1.0 / H)
    h = jnp.where(s >= 0, s, s * pa_ref[0, 0])
    mask = rs_ref[:, 0:1] > 0
    out_ref[...] = jnp.where(mask, h, x_ref[...])


_finalize = pl.pallas_call(
    _final_body,
    grid=(10,),
    in_specs=[
        pl.BlockSpec((NC, 1000, D), lambda i: (0, i, 0)),
        pl.BlockSpec((1000, D), lambda i: (i, 0)),
        pl.BlockSpec((1000, 2), lambda i: (i, 0)),
        pl.BlockSpec((1, 1), lambda i: (0, 0)),
    ],
    out_specs=pl.BlockSpec((1000, D), lambda i: (i, 0)),
    out_shape=jax.ShapeDtypeStruct((N, D), f32),
)


@jax.jit
def _run(x, edge_index, edge_attr, W_rel, b_rel, ln_gamma, ln_beta,
         W_att, b_att, W_w, b_w, prelu_a):
    src = edge_index[0]
    dst = edge_index[1]
    xw, ad, a_s = _node_dense(
        x, W_att[:D, 0:2], W_att[:D, 2:4], W_att[D:2 * D, 0:2],
        W_att[D:2 * D, 2:4],
        W_w[:, :HD2], W_w[:, HD2:],
        b_w[:HD2].reshape(1, HD2), b_w[HD2:].reshape(1, HD2))
    ue = _edge_dense(
        edge_attr, W_rel, b_rel.reshape(1, D), ln_gamma.reshape(1, D),
        ln_beta.reshape(1, D), W_att[2 * D:, 0:2], W_att[2 * D:, 2:4],
        b_att.reshape(1, H))
    p_sc, s_part = _sc1(src, dst, ad.reshape(NC * N * 2),
                        a_s.reshape(NC * N * 2), ue.reshape(NC * E * 2))
    rs = _s_reduce(s_part.reshape(NW, N * 2))
    alpha = _sc_al(dst, p_sc, rs.reshape(NC * N * 2))
    srcx = jnp.stack([src, src + N]).reshape(NC * E)
    acc = _sc_m(srcx, dst, alpha, xw.reshape(NC * N, HD2))
    return _finalize(acc, x, rs[0].reshape(N, 2), prelu_a.reshape(1, 1))


def kernel(x, edge_index, edge_attr, W_rel, b_rel, ln_gamma, ln_beta,
           W_att, b_att, W_w, b_w, prelu_a):
    return _run(x, edge_index, edge_attr, W_rel, b_rel, ln_gamma, ln_beta,
                W_att, b_att, W_w, b_w, prelu_a)


# speedup vs baseline: 1.2729x; 1.2729x over previous
import jax
import jax.numpy as jnp
from jax import lax
from jax.experimental import pallas as pl
from jax.experimental.pallas import tpu as pltpu
from jax.experimental.pallas import tpu_sc as plsc

N = 10000
E = 320000
D = 128
EF = 16
H = 4

NC = 2
NS = 16
NW = NC * NS
ECH = E // NS
CHUNK = E // NW
B3 = 1000
B4 = 40
ACC_R = 10240
ROWS_PT = ACC_R // NS

f32 = jnp.float32
i32 = jnp.int32

_sc_mesh = dict(
    mesh=plsc.VectorSubcoreMesh(
        core_axis_name="c", subcore_axis_name="s", num_cores=NC,
        num_subcores=NS),
    compiler_params=pltpu.CompilerParams(needs_layout_passes=False),
)


def _node_dense_body(x_ref, wd01_ref, wd23_ref, ws01_ref, ws23_ref,
                     ww0_ref, ww1_ref, bw0_ref, bw1_ref,
                     xw_ref, ad_ref, as_ref):
    xb = x_ref[...]
    xw_ref[0] = jnp.dot(xb, ww0_ref[...], preferred_element_type=f32) + bw0_ref[...]
    xw_ref[1] = jnp.dot(xb, ww1_ref[...], preferred_element_type=f32) + bw1_ref[...]
    ad_ref[0] = jnp.dot(xb, wd01_ref[...], preferred_element_type=f32)
    ad_ref[1] = jnp.dot(xb, wd23_ref[...], preferred_element_type=f32)
    as_ref[0] = jnp.dot(xb, ws01_ref[...], preferred_element_type=f32)
    as_ref[1] = jnp.dot(xb, ws23_ref[...], preferred_element_type=f32)


_node_dense = pl.pallas_call(
    _node_dense_body,
    grid=(10,),
    in_specs=[
        pl.BlockSpec((1000, D), lambda i: (i, 0)),
        pl.BlockSpec((D, 2), lambda i: (0, 0)),
        pl.BlockSpec((D, 2), lambda i: (0, 0)),
        pl.BlockSpec((D, 2), lambda i: (0, 0)),
        pl.BlockSpec((D, 2), lambda i: (0, 0)),
        pl.BlockSpec((D, H * D // 2), lambda i: (0, 0)),
        pl.BlockSpec((D, H * D // 2), lambda i: (0, 0)),
        pl.BlockSpec((1, H * D // 2), lambda i: (0, 0)),
        pl.BlockSpec((1, H * D // 2), lambda i: (0, 0)),
    ],
    out_specs=[
        pl.BlockSpec((NC, 1000, H * D // 2), lambda i: (0, i, 0)),
        pl.BlockSpec((NC, 1000, 2), lambda i: (0, i, 0)),
        pl.BlockSpec((NC, 1000, 2), lambda i: (0, i, 0)),
    ],
    out_shape=[
        jax.ShapeDtypeStruct((NC, N, H * D // 2), f32),
        jax.ShapeDtypeStruct((NC, N, 2), f32),
        jax.ShapeDtypeStruct((NC, N, 2), f32),
    ],
)


def _edge_dense_body(ea_ref, wrel_ref, brel_ref, g_ref, b_ref,
                     w301_ref, w323_ref, batt_ref, ue_ref):
    w = jnp.dot(ea_ref[...], wrel_ref[...], preferred_element_type=f32) + brel_ref[...]
    mu = jnp.mean(w, axis=-1, keepdims=True)
    var = jnp.mean((w - mu) ** 2, axis=-1, keepdims=True)
    w = (w - mu) * lax.rsqrt(var + 1e-5) * g_ref[...] + b_ref[...]
    w = jnp.maximum(w, 0.0)
    ue_ref[0] = (jnp.dot(w, w301_ref[...], preferred_element_type=f32)
                 + batt_ref[:, 0:2])
    ue_ref[1] = (jnp.dot(w, w323_ref[...], preferred_element_type=f32)
                 + batt_ref[:, 2:4])


_edge_dense = pl.pallas_call(
    _edge_dense_body,
    grid=(E // 1000,),
    in_specs=[
        pl.BlockSpec((1000, EF), lambda i: (i, 0)),
        pl.BlockSpec((EF, D), lambda i: (0, 0)),
        pl.BlockSpec((1, D), lambda i: (0, 0)),
        pl.BlockSpec((1, D), lambda i: (0, 0)),
        pl.BlockSpec((1, D), lambda i: (0, 0)),
        pl.BlockSpec((D, 2), lambda i: (0, 0)),
        pl.BlockSpec((D, 2), lambda i: (0, 0)),
        pl.BlockSpec((1, H), lambda i: (0, 0)),
    ],
    out_specs=pl.BlockSpec((NC, 1000, 2), lambda i: (0, i, 0)),
    out_shape=jax.ShapeDtypeStruct((NC, E, 2), f32),
)


def _sc_pass1(src_h, dst_h, ad_h, as_h, ue_h, p_h, spart_h,
              ad_t, as_t, src_v, dst_v, ue_v, p_v, s_loc):
    cid = lax.axis_index("c")
    sid = lax.axis_index("s")
    wid = cid * NS + sid
    base = sid * ECH
    iota = lax.iota(i32, 16)
    lane_h = lax.bitwise_and(iota, 1)
    lane_e = lax.shift_right_logical(iota, 1)

    noff = cid * (N * 2)
    eoff = cid * (E * 2)
    pltpu.sync_copy(ad_h.at[pl.ds(noff, N * 2)], ad_t)
    pltpu.sync_copy(as_h.at[pl.ds(noff, N * 2)], as_t)

    def zbody(i, c):
        s_loc[pl.ds(i * 16, 16)] = jnp.zeros((16,), f32)
        return c
    lax.fori_loop(0, N * 2 // 16, zbody, 0)

    def blk(b, c):
        eb = base + b * B3
        pltpu.sync_copy(src_h.at[pl.ds(eb, B3)], src_v)
        pltpu.sync_copy(dst_h.at[pl.ds(eb, B3)], dst_v)
        pltpu.sync_copy(ue_h.at[pl.ds(eoff + eb * 2, B3 * 2)], ue_v)

        def inner(i, c2):
            eidx = i * 8 + lane_e
            dstv = plsc.load_gather(dst_v, [eidx])
            srcv = plsc.load_gather(src_v, [eidx])
            fd = dstv * 2 + lane_h
            gd = plsc.load_gather(ad_t, [fd])
            gs = plsc.load_gather(as_t, [srcv * 2 + lane_h])
            u = gd + gs + ue_v[pl.ds(i * 16, 16)]
            u = jnp.where(u >= 0, u, u * 0.2)
            p = jnp.exp(u)
            p_v[pl.ds(i * 16, 16)] = p
            plsc.addupdate_scatter(s_loc, [fd], p)
            return c2
        lax.fori_loop(0, B3 * 2 // 16, inner, 0)
        pltpu.sync_copy(p_v, p_h.at[pl.ds(eoff + eb * 2, B3 * 2)])
        return c
    lax.fori_loop(0, ECH // B3, blk, 0)
    pltpu.sync_copy(s_loc, spart_h.at[pl.ds(wid * (N * 2), N * 2)])


_sc1 = pl.kernel(
    _sc_pass1,
    out_type=[
        jax.ShapeDtypeStruct((NC * E * 2,), f32),
        jax.ShapeDtypeStruct((NW * N * 2,), f32),
    ],
    scratch_types=[
        pltpu.VMEM((N * 2,), f32),
        pltpu.VMEM((N * 2,), f32),
        pltpu.VMEM((B3,), i32),
        pltpu.VMEM((B3,), i32),
        pltpu.VMEM((B3 * 2,), f32),
        pltpu.VMEM((B3 * 2,), f32),
        pltpu.VMEM((N * 2,), f32),
    ],
    **_sc_mesh,
)


def _s_reduce_body(sp_ref, rs_ref):
    s0 = jnp.sum(sp_ref[0:NS], axis=0)
    s1 = jnp.sum(sp_ref[NS:NW], axis=0)
    s = jnp.stack([s0, s1], axis=0)
    rs_ref[...] = jnp.where(s > 0, 1.0 / s, 0.0)


_s_reduce = pl.pallas_call(
    _s_reduce_body,
    grid=(1,),
    in_specs=[pl.BlockSpec((NW, N * 2), lambda i: (0, 0))],
    out_specs=pl.BlockSpec((NC, N * 2), lambda i: (0, 0)),
    out_shape=jax.ShapeDtypeStruct((NC, N * 2), f32),
)


def _sc_alpha(dst_h, p_h, rs_h, al_h, rs_t, dst_v, p_v, al_v):
    cid = lax.axis_index("c")
    sid = lax.axis_index("s")
    base = sid * ECH
    iota = lax.iota(i32, 16)
    lane_h = lax.bitwise_and(iota, 1)
    lane_e = lax.shift_right_logical(iota, 1)

    noff = cid * (N * 2)
    eoff = cid * (E * 2)
    pltpu.sync_copy(rs_h.at[pl.ds(noff, N * 2)], rs_t)

    def blk(b, c):
        eb = base + b * B3
        pltpu.sync_copy(dst_h.at[pl.ds(eb, B3)], dst_v)
        pltpu.sync_copy(p_h.at[pl.ds(eoff + eb * 2, B3 * 2)], p_v)

        def inner(i, c2):
            eidx = i * 8 + lane_e
            dstv = plsc.load_gather(dst_v, [eidx])
            rsv = plsc.load_gather(rs_t, [dstv * 2 + lane_h])
            al_v[pl.ds(i * 16, 16)] = p_v[pl.ds(i * 16, 16)] * rsv
            return c2
        lax.fori_loop(0, B3 * 2 // 16, inner, 0)
        pltpu.sync_copy(al_v, al_h.at[pl.ds(eoff + eb * 2, B3 * 2)])
        return c
    lax.fori_loop(0, ECH // B3, blk, 0)


_sc_al = pl.kernel(
    _sc_alpha,
    out_type=jax.ShapeDtypeStruct((NC * E * 2,), f32),
    scratch_types=[
        pltpu.VMEM((N * 2,), f32),
        pltpu.VMEM((B3,), i32),
        pltpu.VMEM((B3 * 2,), f32),
        pltpu.VMEM((B3 * 2,), f32),
    ],
    **_sc_mesh,
)


HD2 = H * D // 2
FH = D // 2
SUP = 2000
NSUP = ECH // SUP
BPS = SUP // B4


NBUF = 2


def _sc_msg(srcx_h, dst_h, al_h, xw_h, acc_h,
            idx_sb, dst_sb, al0_sb,
            xw_bufs, m_bufs, acc_sh, g_sems, s_sem):
    cid = lax.axis_index("c")
    sid = lax.axis_index("s")
    base = sid * ECH

    for i in range(B4):
        for c in range(D // 16):
            m_bufs[0][i, pl.ds(c * 16, 16)] = jnp.zeros((16,), f32)
    row0 = sid * ROWS_PT

    def zcp(j, c):
        pltpu.sync_copy(m_bufs[0], acc_sh.at[pl.ds(row0 + j * B4, B4)])
        return c
    lax.fori_loop(0, ROWS_PT // B4, zcp, 0)
    plsc.subcore_barrier()

    def eloop(blk, xwv, mv):
        def eb(e, c):
            le2 = (blk * B4 + e) * 2
            a0 = plsc.load_gather(al0_sb, [jnp.full((16,), le2, i32)])
            a1 = plsc.load_gather(al0_sb, [jnp.full((16,), le2 + 1, i32)])
            for c4 in range(D // 16):
                v = a0 * xwv[e, pl.ds(c4 * 16, 16)]
                v = v + a1 * xwv[e, pl.ds(D + c4 * 16, 16)]
                mv[e, pl.ds(c4 * 16, 16)] = v
            return c
        lax.fori_loop(0, B4, eb, 0)

    def super_body(s, c):
        off = base + s * SUP
        pltpu.sync_copy(al_h.at[pl.ds(cid * (E * 2) + off * 2, SUP * 2)],
                        al0_sb)
        pltpu.sync_copy(srcx_h.at[cid, sid, s], idx_sb)
        pltpu.sync_copy(dst_h.at[sid, s], dst_sb)

        def group(gr, c2):
            b0 = gr * NBUF
            gds = [
                pltpu.async_copy(
                    xw_h.at[idx_sb.at[b0 + j]], xw_bufs[j], g_sems[j])
                for j in range(NBUF)
            ]
            sds = []
            for j in range(NBUF):
                gds[j].wait()
                eloop(b0 + j, xw_bufs[j], m_bufs[j])
                sds.append(pltpu.async_copy(
                    m_bufs[j], acc_sh.at[dst_sb.at[b0 + j]], s_sem,
                    add=True))
            for d in sds:
                d.wait()
            return c2
        lax.fori_loop(0, BPS // NBUF, group, 0)
        return c
    lax.fori_loop(0, NSUP, super_body, 0)
    plsc.subcore_barrier()
    pltpu.sync_copy(acc_sh.at[pl.ds(row0, ROWS_PT)],
                    acc_h.at[cid, pl.ds(row0, ROWS_PT)])


_sc_m = pl.kernel(
    _sc_msg,
    out_type=jax.ShapeDtypeStruct((NC, ACC_R, D), f32),
    scratch_types=[
        pltpu.VMEM((BPS, B4), i32),
        pltpu.VMEM((BPS, B4), i32),
        pltpu.VMEM((SUP * 2,), f32),
        [pltpu.VMEM((B4, HD2), f32)] * NBUF,
        [pltpu.VMEM((B4, D), f32)] * NBUF,
        pltpu.VMEM_SHARED((ACC_R, D), f32),
        [pltpu.SemaphoreType.DMA] * NBUF,
        pltpu.SemaphoreType.DMA,
    ],
    **_sc_mesh,
)


def _final_body(acc_ref, x_ref, rs_ref, pa_ref, out_ref):
    s = (acc_ref[0] + acc_ref[1]) * (1.0 / H)
    h = jnp.where(s >= 0, s, s * pa_ref[0, 0])
    mask = rs_ref[:, 0:1] > 0
    out_ref[...] = jnp.where(mask, h, x_ref[...])


_finalize = pl.pallas_call(
    _final_body,
    grid=(10,),
    in_specs=[
        pl.BlockSpec((NC, 1000, D), lambda i: (0, i, 0)),
        pl.BlockSpec((1000, D), lambda i: (i, 0)),
        pl.BlockSpec((1000, 2), lambda i: (i, 0)),
        pl.BlockSpec((1, 1), lambda i: (0, 0)),
    ],
    out_specs=pl.BlockSpec((1000, D), lambda i: (i, 0)),
    out_shape=jax.ShapeDtypeStruct((N, D), f32),
)


@jax.jit
def _run(x, edge_index, edge_attr, W_rel, b_rel, ln_gamma, ln_beta,
         W_att, b_att, W_w, b_w, prelu_a):
    src = edge_index[0]
    dst = edge_index[1]
    xw, ad, a_s = _node_dense(
        x, W_att[:D, 0:2], W_att[:D, 2:4], W_att[D:2 * D, 0:2],
        W_att[D:2 * D, 2:4],
        W_w[:, :HD2], W_w[:, HD2:],
        b_w[:HD2].reshape(1, HD2), b_w[HD2:].reshape(1, HD2))
    ue = _edge_dense(
        edge_attr, W_rel, b_rel.reshape(1, D), ln_gamma.reshape(1, D),
        ln_beta.reshape(1, D), W_att[2 * D:, 0:2], W_att[2 * D:, 2:4],
        b_att.reshape(1, H))
    p_sc, s_part = _sc1(src, dst, ad.reshape(NC * N * 2),
                        a_s.reshape(NC * N * 2), ue.reshape(NC * E * 2))
    rs = _s_reduce(s_part.reshape(NW, N * 2))
    alpha = _sc_al(dst, p_sc, rs.reshape(NC * N * 2))
    srcx = jnp.stack([src, src + N]).reshape(NC, NS, NSUP, BPS, B4)
    acc = _sc_m(srcx, dst.reshape(NS, NSUP, BPS, B4), alpha,
                xw.reshape(NC * N, HD2))
    return _finalize(acc, x, rs[0].reshape(N, 2), prelu_a.reshape(1, 1))


def kernel(x, edge_index, edge_attr, W_rel, b_rel, ln_gamma, ln_beta,
           W_att, b_att, W_w, b_w, prelu_a):
    return _run(x, edge_index, edge_attr, W_rel, b_rel, ln_gamma, ln_beta,
                W_att, b_att, W_w, b_w, prelu_a)


# speedup vs baseline: 1.4983x; 1.1771x over previous
import jax
import jax.numpy as jnp
from jax import lax
from jax.experimental import pallas as pl
from jax.experimental.pallas import tpu as pltpu
from jax.experimental.pallas import tpu_sc as plsc

N = 10000
E = 320000
D = 128
EF = 16
H = 4

NC = 2
NS = 16
NW = NC * NS
ECH = E // NS
CHUNK = E // NW
B3 = 1000
B4 = 40
ACC_R = 10240
ROWS_PT = ACC_R // NS

f32 = jnp.float32
i32 = jnp.int32

_sc_mesh = dict(
    mesh=plsc.VectorSubcoreMesh(
        core_axis_name="c", subcore_axis_name="s", num_cores=NC,
        num_subcores=NS),
    compiler_params=pltpu.CompilerParams(needs_layout_passes=False),
)


def _node_dense_body(x_ref, wd01_ref, wd23_ref, ws01_ref, ws23_ref,
                     ww0_ref, ww1_ref, bw0_ref, bw1_ref,
                     xw_ref, ad_ref, as_ref):
    xb = x_ref[...]
    xw_ref[0] = jnp.dot(xb, ww0_ref[...], preferred_element_type=f32) + bw0_ref[...]
    xw_ref[1] = jnp.dot(xb, ww1_ref[...], preferred_element_type=f32) + bw1_ref[...]
    ad_ref[0] = jnp.dot(xb, wd01_ref[...], preferred_element_type=f32)
    ad_ref[1] = jnp.dot(xb, wd23_ref[...], preferred_element_type=f32)
    as_ref[0] = jnp.dot(xb, ws01_ref[...], preferred_element_type=f32)
    as_ref[1] = jnp.dot(xb, ws23_ref[...], preferred_element_type=f32)


_node_dense = pl.pallas_call(
    _node_dense_body,
    grid=(10,),
    in_specs=[
        pl.BlockSpec((1000, D), lambda i: (i, 0)),
        pl.BlockSpec((D, 2), lambda i: (0, 0)),
        pl.BlockSpec((D, 2), lambda i: (0, 0)),
        pl.BlockSpec((D, 2), lambda i: (0, 0)),
        pl.BlockSpec((D, 2), lambda i: (0, 0)),
        pl.BlockSpec((D, H * D // 2), lambda i: (0, 0)),
        pl.BlockSpec((D, H * D // 2), lambda i: (0, 0)),
        pl.BlockSpec((1, H * D // 2), lambda i: (0, 0)),
        pl.BlockSpec((1, H * D // 2), lambda i: (0, 0)),
    ],
    out_specs=[
        pl.BlockSpec((NC, 1000, H * D // 2), lambda i: (0, i, 0)),
        pl.BlockSpec((NC, 1000, 2), lambda i: (0, i, 0)),
        pl.BlockSpec((NC, 1000, 2), lambda i: (0, i, 0)),
    ],
    out_shape=[
        jax.ShapeDtypeStruct((NC, N, H * D // 2), f32),
        jax.ShapeDtypeStruct((NC, N, 2), f32),
        jax.ShapeDtypeStruct((NC, N, 2), f32),
    ],
)


def _edge_dense_body(ea_ref, wrel_ref, brel_ref, g_ref, b_ref,
                     w301_ref, w323_ref, batt_ref, ue_ref):
    w = jnp.dot(ea_ref[...], wrel_ref[...], preferred_element_type=f32) + brel_ref[...]
    mu = jnp.mean(w, axis=-1, keepdims=True)
    var = jnp.mean((w - mu) ** 2, axis=-1, keepdims=True)
    w = (w - mu) * lax.rsqrt(var + 1e-5) * g_ref[...] + b_ref[...]
    w = jnp.maximum(w, 0.0)
    ue_ref[0] = (jnp.dot(w, w301_ref[...], preferred_element_type=f32)
                 + batt_ref[:, 0:2])
    ue_ref[1] = (jnp.dot(w, w323_ref[...], preferred_element_type=f32)
                 + batt_ref[:, 2:4])


_edge_dense = pl.pallas_call(
    _edge_dense_body,
    grid=(E // 1000,),
    in_specs=[
        pl.BlockSpec((1000, EF), lambda i: (i, 0)),
        pl.BlockSpec((EF, D), lambda i: (0, 0)),
        pl.BlockSpec((1, D), lambda i: (0, 0)),
        pl.BlockSpec((1, D), lambda i: (0, 0)),
        pl.BlockSpec((1, D), lambda i: (0, 0)),
        pl.BlockSpec((D, 2), lambda i: (0, 0)),
        pl.BlockSpec((D, 2), lambda i: (0, 0)),
        pl.BlockSpec((1, H), lambda i: (0, 0)),
    ],
    out_specs=pl.BlockSpec((NC, 1000, 2), lambda i: (0, i, 0)),
    out_shape=jax.ShapeDtypeStruct((NC, E, 2), f32),
)


def _sc_pass1(src_h, dst_h, ad_h, as_h, ue_h, p_h, spart_h,
              ad_t, as_t, src_v, dst_v, ue_v, p_v, s_loc):
    cid = lax.axis_index("c")
    sid = lax.axis_index("s")
    wid = cid * NS + sid
    base = sid * ECH
    iota = lax.iota(i32, 16)
    lane_h = lax.bitwise_and(iota, 1)
    lane_e = lax.shift_right_logical(iota, 1)

    noff = cid * (N * 2)
    eoff = cid * (E * 2)
    pltpu.sync_copy(ad_h.at[pl.ds(noff, N * 2)], ad_t)
    pltpu.sync_copy(as_h.at[pl.ds(noff, N * 2)], as_t)

    def zbody(i, c):
        s_loc[pl.ds(i * 16, 16)] = jnp.zeros((16,), f32)
        return c
    lax.fori_loop(0, N * 2 // 16, zbody, 0)

    def blk(b, c):
        eb = base + b * B3
        pltpu.sync_copy(src_h.at[pl.ds(eb, B3)], src_v)
        pltpu.sync_copy(dst_h.at[pl.ds(eb, B3)], dst_v)
        pltpu.sync_copy(ue_h.at[pl.ds(eoff + eb * 2, B3 * 2)], ue_v)

        def inner(i, c2):
            eidx = i * 8 + lane_e
            dstv = plsc.load_gather(dst_v, [eidx])
            srcv = plsc.load_gather(src_v, [eidx])
            fd = dstv * 2 + lane_h
            gd = plsc.load_gather(ad_t, [fd])
            gs = plsc.load_gather(as_t, [srcv * 2 + lane_h])
            u = gd + gs + ue_v[pl.ds(i * 16, 16)]
            u = jnp.where(u >= 0, u, u * 0.2)
            p = jnp.exp(u)
            p_v[pl.ds(i * 16, 16)] = p
            plsc.addupdate_scatter(s_loc, [fd], p)
            return c2
        lax.fori_loop(0, B3 * 2 // 16, inner, 0)
        pltpu.sync_copy(p_v, p_h.at[pl.ds(eoff + eb * 2, B3 * 2)])
        return c
    lax.fori_loop(0, ECH // B3, blk, 0)
    pltpu.sync_copy(s_loc, spart_h.at[pl.ds(wid * (N * 2), N * 2)])


_sc1 = pl.kernel(
    _sc_pass1,
    out_type=[
        jax.ShapeDtypeStruct((NC * E * 2,), f32),
        jax.ShapeDtypeStruct((NW * N * 2,), f32),
    ],
    scratch_types=[
        pltpu.VMEM((N * 2,), f32),
        pltpu.VMEM((N * 2,), f32),
        pltpu.VMEM((B3,), i32),
        pltpu.VMEM((B3,), i32),
        pltpu.VMEM((B3 * 2,), f32),
        pltpu.VMEM((B3 * 2,), f32),
        pltpu.VMEM((N * 2,), f32),
    ],
    **_sc_mesh,
)


def _s_reduce_body(sp_ref, rs_ref):
    s0 = jnp.sum(sp_ref[0:NS], axis=0)
    s1 = jnp.sum(sp_ref[NS:NW], axis=0)
    s = jnp.stack([s0, s1], axis=0)
    rs_ref[...] = jnp.where(s > 0, 1.0 / s, 0.0)


_s_reduce = pl.pallas_call(
    _s_reduce_body,
    grid=(1,),
    in_specs=[pl.BlockSpec((NW, N * 2), lambda i: (0, 0))],
    out_specs=pl.BlockSpec((NC, N * 2), lambda i: (0, 0)),
    out_shape=jax.ShapeDtypeStruct((NC, N * 2), f32),
)


def _sc_alpha(dst_h, p_h, rs_h, al_h, rs_t, dst_v, p_v, al_v):
    cid = lax.axis_index("c")
    sid = lax.axis_index("s")
    base = sid * ECH
    iota = lax.iota(i32, 16)
    lane_h = lax.bitwise_and(iota, 1)
    lane_e = lax.shift_right_logical(iota, 1)

    noff = cid * (N * 2)
    eoff = cid * (E * 2)
    pltpu.sync_copy(rs_h.at[pl.ds(noff, N * 2)], rs_t)

    def blk(b, c):
        eb = base + b * B3
        pltpu.sync_copy(dst_h.at[pl.ds(eb, B3)], dst_v)
        pltpu.sync_copy(p_h.at[pl.ds(eoff + eb * 2, B3 * 2)], p_v)

        def inner(i, c2):
            eidx = i * 8 + lane_e
            dstv = plsc.load_gather(dst_v, [eidx])
            rsv = plsc.load_gather(rs_t, [dstv * 2 + lane_h])
            al_v[pl.ds(i * 16, 16)] = p_v[pl.ds(i * 16, 16)] * rsv
            return c2
        lax.fori_loop(0, B3 * 2 // 16, inner, 0)
        pltpu.sync_copy(al_v, al_h.at[pl.ds(eoff + eb * 2, B3 * 2)])
        return c
    lax.fori_loop(0, ECH // B3, blk, 0)


_sc_al = pl.kernel(
    _sc_alpha,
    out_type=jax.ShapeDtypeStruct((NC * E * 2,), f32),
    scratch_types=[
        pltpu.VMEM((N * 2,), f32),
        pltpu.VMEM((B3,), i32),
        pltpu.VMEM((B3 * 2,), f32),
        pltpu.VMEM((B3 * 2,), f32),
    ],
    **_sc_mesh,
)


HD2 = H * D // 2
FH = D // 2
SUP = 2000
NSUP = ECH // SUP
BPS = SUP // B4


NBUF = 2


def _sc_msg(srcx_h, dst_h, al_h, xw_h, acc_h,
            idx_sb, dst_sb, al0_sb,
            xw_bufs, m_bufs, acc_sh, g_sems, s_sems):
    cid = lax.axis_index("c")
    sid = lax.axis_index("s")
    base = sid * ECH

    for i in range(B4):
        for c in range(D // 16):
            m_bufs[0][i, pl.ds(c * 16, 16)] = jnp.zeros((16,), f32)
    row0 = sid * ROWS_PT

    def zcp(j, c):
        pltpu.sync_copy(m_bufs[0], acc_sh.at[pl.ds(row0 + j * B4, B4)])
        return c
    lax.fori_loop(0, ROWS_PT // B4, zcp, 0)
    plsc.subcore_barrier()

    def eloop(blk, xwv, mv):
        def eb(e, c):
            le2 = (blk * B4 + e) * 2
            a0 = plsc.load_gather(al0_sb, [jnp.full((16,), le2, i32)])
            a1 = plsc.load_gather(al0_sb, [jnp.full((16,), le2 + 1, i32)])
            for c4 in range(D // 16):
                v = a0 * xwv[e, pl.ds(c4 * 16, 16)]
                v = v + a1 * xwv[e, pl.ds(D + c4 * 16, 16)]
                mv[e, pl.ds(c4 * 16, 16)] = v
            return c
        lax.fori_loop(0, B4, eb, 0)

    def super_body(s, c):
        off = base + s * SUP
        pltpu.sync_copy(al_h.at[pl.ds(cid * (E * 2) + off * 2, SUP * 2)],
                        al0_sb)
        pltpu.sync_copy(srcx_h.at[cid, sid, s], idx_sb)
        pltpu.sync_copy(dst_h.at[sid, s], dst_sb)

        def wait_g(j):
            pltpu.make_async_copy(
                xw_h.at[idx_sb.at[0]], xw_bufs[j], g_sems[j]).wait()

        def wait_s(j):
            pltpu.make_async_copy(
                m_bufs[j], acc_sh.at[dst_sb.at[0]], s_sems[j]).wait()

        pltpu.async_copy(xw_h.at[idx_sb.at[0]], xw_bufs[0], g_sems[0])

        def pair(g, c2):
            blkA = 2 * g
            wait_g(0)
            pltpu.async_copy(
                xw_h.at[idx_sb.at[blkA + 1]], xw_bufs[1], g_sems[1])

            @pl.when(g > 0)
            def _():
                wait_s(0)
            eloop(blkA, xw_bufs[0], m_bufs[0])
            pltpu.async_copy(
                m_bufs[0], acc_sh.at[dst_sb.at[blkA]], s_sems[0], add=True)

            wait_g(1)

            @pl.when(g < BPS // 2 - 1)
            def _():
                pltpu.async_copy(
                    xw_h.at[idx_sb.at[blkA + 2]], xw_bufs[0], g_sems[0])

            @pl.when(g > 0)
            def _():
                wait_s(1)
            eloop(blkA + 1, xw_bufs[1], m_bufs[1])
            pltpu.async_copy(
                m_bufs[1], acc_sh.at[dst_sb.at[blkA + 1]], s_sems[1],
                add=True)
            return c2
        lax.fori_loop(0, BPS // 2, pair, 0)
        wait_s(0)
        wait_s(1)
        return c
    lax.fori_loop(0, NSUP, super_body, 0)
    plsc.subcore_barrier()
    pltpu.sync_copy(acc_sh.at[pl.ds(row0, ROWS_PT)],
                    acc_h.at[cid, pl.ds(row0, ROWS_PT)])


_sc_m = pl.kernel(
    _sc_msg,
    out_type=jax.ShapeDtypeStruct((NC, ACC_R, D), f32),
    scratch_types=[
        pltpu.VMEM((BPS, B4), i32),
        pltpu.VMEM((BPS, B4), i32),
        pltpu.VMEM((SUP * 2,), f32),
        [pltpu.VMEM((B4, HD2), f32)] * NBUF,
        [pltpu.VMEM((B4, D), f32)] * NBUF,
        pltpu.VMEM_SHARED((ACC_R, D), f32),
        [pltpu.SemaphoreType.DMA] * NBUF,
        [pltpu.SemaphoreType.DMA] * NBUF,
    ],
    **_sc_mesh,
)


def _final_body(acc_ref, x_ref, rs_ref, pa_ref, out_ref):
    s = (acc_ref[0] + acc_ref[1]) * (1.0 / H)
    h = jnp.where(s >= 0, s, s * pa_ref[0, 0])
    mask = rs_ref[:, 0:1] > 0
    out_ref[...] = jnp.where(mask, h, x_ref[...])


_finalize = pl.pallas_call(
    _final_body,
    grid=(10,),
    in_specs=[
        pl.BlockSpec((NC, 1000, D), lambda i: (0, i, 0)),
        pl.BlockSpec((1000, D), lambda i: (i, 0)),
        pl.BlockSpec((1000, 2), lambda i: (i, 0)),
        pl.BlockSpec((1, 1), lambda i: (0, 0)),
    ],
    out_specs=pl.BlockSpec((1000, D), lambda i: (i, 0)),
    out_shape=jax.ShapeDtypeStruct((N, D), f32),
)


@jax.jit
def _run(x, edge_index, edge_attr, W_rel, b_rel, ln_gamma, ln_beta,
         W_att, b_att, W_w, b_w, prelu_a):
    src = edge_index[0]
    dst = edge_index[1]
    xw, ad, a_s = _node_dense(
        x, W_att[:D, 0:2], W_att[:D, 2:4], W_att[D:2 * D, 0:2],
        W_att[D:2 * D, 2:4],
        W_w[:, :HD2], W_w[:, HD2:],
        b_w[:HD2].reshape(1, HD2), b_w[HD2:].reshape(1, HD2))
    ue = _edge_dense(
        edge_attr, W_rel, b_rel.reshape(1, D), ln_gamma.reshape(1, D),
        ln_beta.reshape(1, D), W_att[2 * D:, 0:2], W_att[2 * D:, 2:4],
        b_att.reshape(1, H))
    p_sc, s_part = _sc1(src, dst, ad.reshape(NC * N * 2),
                        a_s.reshape(NC * N * 2), ue.reshape(NC * E * 2))
    rs = _s_reduce(s_part.reshape(NW, N * 2))
    alpha = _sc_al(dst, p_sc, rs.reshape(NC * N * 2))
    srcx = jnp.stack([src, src + N]).reshape(NC, NS, NSUP, BPS, B4)
    acc = _sc_m(srcx, dst.reshape(NS, NSUP, BPS, B4), alpha,
                xw.reshape(NC * N, HD2))
    return _finalize(acc, x, rs[0].reshape(N, 2), prelu_a.reshape(1, 1))


def kernel(x, edge_index, edge_attr, W_rel, b_rel, ln_gamma, ln_beta,
           W_att, b_att, W_w, b_w, prelu_a):
    return _run(x, edge_index, edge_attr, W_rel, b_rel, ln_gamma, ln_beta,
                W_att, b_att, W_w, b_w, prelu_a)
